# Initial kernel scaffold; baseline (speedup 1.0000x reference)
import jax
import jax.numpy as jnp
from jax.experimental import pallas as pl


def kernel(x, edge_index, edge_attr, batch, W1, b1, W2, b2, W3, b3, Wl1, bl1, Wl2, bl2):
    raise NotImplementedError("write your pallas kernel here")


# speedup vs baseline: 8.6508x; 8.6508x over previous
import functools

import jax
import jax.numpy as jnp
from jax import lax
from jax.experimental import pallas as pl
from jax.experimental.pallas import tpu as pltpu
from jax.experimental.pallas import tpu_sc as plsc

N = 10000
E = 320000
F = 128
G = 128
NCLS = 10
NPAD = 10240

NC, NS, L = 2, 16, 16
NW = NC * NS
EW = E // NW
CH = 128
NFULL = EW // CH
TAIL = EW - NFULL * CH
RPS = NPAD // NS

_mesh = plsc.VectorSubcoreMesh(core_axis_name="c", subcore_axis_name="s")


@functools.partial(
    pl.kernel,
    out_type=jax.ShapeDtypeStruct((NC, NPAD), jnp.float32),
    mesh=_mesh,
    scratch_types=[
        pltpu.VMEM((1, CH), jnp.int32),
        pltpu.VMEM((1, CH), jnp.float32),
        pltpu.VMEM((1, TAIL), jnp.int32),
        pltpu.VMEM((1, TAIL), jnp.float32),
        pltpu.VMEM((RPS,), jnp.float32),
        pltpu.VMEM_SHARED((NPAD,), jnp.float32),
    ],
)
def _deg(row, ea, out, idx_v, w_v, idx_t, w_t, z_v, acc):
    cid = lax.axis_index("c")
    sid = lax.axis_index("s")
    wid = sid * NC + cid

    def zb(i, _):
        z_v[pl.ds(i * L, L)] = jnp.zeros((L,), jnp.float32)
        return 0

    lax.fori_loop(0, RPS // L, zb, 0)
    pltpu.sync_copy(z_v, acc.at[pl.ds(sid * RPS, RPS)])
    plsc.subcore_barrier()

    wbase = wid * EW

    def body(k, _):
        base = wbase + k * CH
        pltpu.sync_copy(row.at[pl.ds(base, CH)], idx_v.at[0])
        pltpu.sync_copy(ea.at[pl.ds(base, CH)], w_v.at[0])
        pltpu.sync_copy(w_v.at[0], acc.at[idx_v.at[0]], add=True)
        return 0

    lax.fori_loop(0, NFULL, body, 0)

    tbase = wbase + NFULL * CH
    pltpu.sync_copy(row.at[pl.ds(tbase, TAIL)], idx_t.at[0])
    pltpu.sync_copy(ea.at[pl.ds(tbase, TAIL)], w_t.at[0])
    pltpu.sync_copy(w_t.at[0], acc.at[idx_t.at[0]], add=True)

    plsc.subcore_barrier()
    pltpu.sync_copy(acc.at[pl.ds(sid * RPS, RPS)],
                    out.at[cid, pl.ds(sid * RPS, RPS)])


@functools.partial(
    pl.kernel,
    out_type=jax.ShapeDtypeStruct((NC, NPAD, F), jnp.float32),
    mesh=_mesh,
    scratch_types=[
        pltpu.VMEM((1, CH), jnp.int32),
        pltpu.VMEM((1, CH), jnp.int32),
        pltpu.VMEM((1, CH), jnp.float32),
        pltpu.VMEM((1, TAIL), jnp.int32),
        pltpu.VMEM((1, TAIL), jnp.int32),
        pltpu.VMEM((1, TAIL), jnp.float32),
        pltpu.VMEM((CH, F), jnp.float32),
        pltpu.VMEM_SHARED((NPAD, F), jnp.float32),
        pltpu.SemaphoreType.DMA,
    ],
)
def _prop(y, erow, ecol, ea, out, ri, ci, wv, rit, cit, wvt, rows, acc, sem):
    cid = lax.axis_index("c")
    sid = lax.axis_index("s")
    wid = sid * NC + cid

    def zb(e, _):
        for j in range(F // L):
            rows[e, pl.ds(j * L, L)] = jnp.zeros((L,), jnp.float32)
        return 0

    lax.fori_loop(0, CH, zb, 0)
    for r in range(RPS // CH):
        pltpu.sync_copy(rows, acc.at[pl.ds(sid * RPS + r * CH, CH)])
    plsc.subcore_barrier()

    wbase = wid * EW

    def body(k, _):
        base = wbase + k * CH
        pltpu.sync_copy(erow.at[pl.ds(base, CH)], ri.at[0])
        pltpu.sync_copy(ecol.at[pl.ds(base, CH)], ci.at[0])
        pltpu.sync_copy(ea.at[pl.ds(base, CH)], wv.at[0])
        pltpu.async_copy(y.at[ri.at[0]], rows, sem).wait()

        def sc(g, _):
            wvec = wv[0, pl.ds(g * L, L)]
            for t in range(L):
                e = g * L + t
                s = wvec[t]
                for j in range(F // L):
                    rows[e, pl.ds(j * L, L)] = rows[e, pl.ds(j * L, L)] * s
            return 0

        lax.fori_loop(0, CH // L, sc, 0)
        pltpu.sync_copy(rows, acc.at[ci.at[0]], add=True)
        return 0

    lax.fori_loop(0, NFULL, body, 0)

    tbase = wbase + NFULL * CH
    pltpu.sync_copy(erow.at[pl.ds(tbase, TAIL)], rit.at[0])
    pltpu.sync_copy(ecol.at[pl.ds(tbase, TAIL)], cit.at[0])
    pltpu.sync_copy(ea.at[pl.ds(tbase, TAIL)], wvt.at[0])
    pltpu.async_copy(y.at[rit.at[0]], rows.at[pl.ds(0, TAIL)], sem).wait()

    wvec_t = wvt[0]
    for t in range(TAIL):
        s = wvec_t[t]
        for j in range(F // L):
            rows[t, pl.ds(j * L, L)] = rows[t, pl.ds(j * L, L)] * s
    pltpu.sync_copy(rows.at[pl.ds(0, TAIL)], acc.at[cit.at[0]], add=True)

    plsc.subcore_barrier()
    for r in range(RPS // CH):
        off = sid * RPS + r * CH
        pltpu.sync_copy(acc.at[pl.ds(off, CH)], out.at[cid, pl.ds(off, CH)])


BR = 1024
NBLK = NPAD // BR


def _tca_body(x_ref, dp_ref, w10_ref, dis_ref, y0_ref, t0_ref):
    dp = dp_ref[...]
    deg = dp[0:1, :] + dp[1:2, :]
    dis = jnp.where(deg > 0, lax.rsqrt(jnp.where(deg > 0, deg, 1.0)), 0.0)
    dcol = jnp.transpose(dis)
    dis_ref[...] = dcol
    xb = x_ref[...]
    y0_ref[...] = dcol * xb
    t0_ref[...] = jnp.dot(xb, w10_ref[...], preferred_element_type=jnp.float32)


_tca = pl.pallas_call(
    _tca_body,
    grid=(NBLK,),
    in_specs=[
        pl.BlockSpec((BR, F), lambda i: (i, 0)),
        pl.BlockSpec((NC, BR), lambda i: (0, i)),
        pl.BlockSpec((F, F), lambda i: (0, 0)),
    ],
    out_specs=[
        pl.BlockSpec((BR, 1), lambda i: (i, 0)),
        pl.BlockSpec((BR, F), lambda i: (i, 0)),
        pl.BlockSpec((BR, F), lambda i: (i, 0)),
    ],
    out_shape=[
        jax.ShapeDtypeStruct((NPAD, 1), jnp.float32),
        jax.ShapeDtypeStruct((NPAD, F), jnp.float32),
        jax.ShapeDtypeStruct((NPAD, F), jnp.float32),
    ],
)


def _tcb_body(t_ref, p_ref, d_ref, w1_ref, b_ref, wn_ref, y_ref, tn_ref):
    q = p_ref[0] + p_ref[1]
    d = d_ref[...]
    tx1 = -(d * q)
    h = (t_ref[...]
         + jnp.dot(tx1, w1_ref[...], preferred_element_type=jnp.float32)
         + b_ref[...])
    h = jnp.maximum(h, 0.0)
    y_ref[...] = d * h
    tn_ref[...] = jnp.dot(h, wn_ref[...], preferred_element_type=jnp.float32)


_tcb = pl.pallas_call(
    _tcb_body,
    grid=(NBLK,),
    in_specs=[
        pl.BlockSpec((BR, F), lambda i: (i, 0)),
        pl.BlockSpec((NC, BR, F), lambda i: (0, i, 0)),
        pl.BlockSpec((BR, 1), lambda i: (i, 0)),
        pl.BlockSpec((F, F), lambda i: (0, 0)),
        pl.BlockSpec((1, F), lambda i: (0, 0)),
        pl.BlockSpec((F, F), lambda i: (0, 0)),
    ],
    out_specs=[
        pl.BlockSpec((BR, F), lambda i: (i, 0)),
        pl.BlockSpec((BR, F), lambda i: (i, 0)),
    ],
    out_shape=[
        jax.ShapeDtypeStruct((NPAD, F), jnp.float32),
        jax.ShapeDtypeStruct((NPAD, F), jnp.float32),
    ],
)


def _tcc_body(t_ref, p_ref, d_ref, w31_ref, b3_ref, bat_ref,
              wl1_ref, bl1_ref, wl2_ref, bl2_ref, out_ref, pooled, cnt):
    i = pl.program_id(0)

    @pl.when(i == 0)
    def _():
        pooled[...] = jnp.zeros_like(pooled)
        cnt[...] = jnp.zeros_like(cnt)

    q = p_ref[0] + p_ref[1]
    h3 = (t_ref[...]
          + jnp.dot(-(d_ref[...] * q), w31_ref[...],
                    preferred_element_type=jnp.float32)
          + b3_ref[...])
    bat = bat_ref[...]
    gids = lax.broadcasted_iota(jnp.int32, (G, BR), 0)
    onehot = jnp.where(gids == bat, 1.0, 0.0)
    pooled[...] += jnp.dot(onehot, h3, preferred_element_type=jnp.float32)
    cnt[...] += jnp.sum(onehot, axis=1, keepdims=True)

    @pl.when(i == pl.num_programs(0) - 1)
    def _():
        pm = pooled[...] / jnp.maximum(cnt[...], 1.0)
        z = jnp.maximum(
            jnp.dot(pm, wl1_ref[...], preferred_element_type=jnp.float32)
            + bl1_ref[...], 0.0)
        out_ref[...] = (jnp.dot(z, wl2_ref[...],
                                preferred_element_type=jnp.float32)
                        + bl2_ref[...])


_tcc = pl.pallas_call(
    _tcc_body,
    grid=(NBLK,),
    in_specs=[
        pl.BlockSpec((BR, F), lambda i: (i, 0)),
        pl.BlockSpec((NC, BR, F), lambda i: (0, i, 0)),
        pl.BlockSpec((BR, 1), lambda i: (i, 0)),
        pl.BlockSpec((F, F), lambda i: (0, 0)),
        pl.BlockSpec((1, F), lambda i: (0, 0)),
        pl.BlockSpec((1, BR), lambda i: (0, i)),
        pl.BlockSpec((F, 32), lambda i: (0, 0)),
        pl.BlockSpec((1, 32), lambda i: (0, 0)),
        pl.BlockSpec((32, NCLS), lambda i: (0, 0)),
        pl.BlockSpec((1, NCLS), lambda i: (0, 0)),
    ],
    out_specs=pl.BlockSpec((G, NCLS), lambda i: (0, 0)),
    out_shape=jax.ShapeDtypeStruct((G, NCLS), jnp.float32),
    scratch_shapes=[
        pltpu.VMEM((G, F), jnp.float32),
        pltpu.VMEM((G, 1), jnp.float32),
    ],
)


def kernel(x, edge_index, edge_attr, batch,
           W1, b1, W2, b2, W3, b3, Wl1, bl1, Wl2, bl2):
    xp = jnp.pad(x, ((0, NPAD - N), (0, 0)))
    batp = jnp.pad(batch, (0, NPAD - N), constant_values=G).reshape(1, NPAD)

    erow = edge_index[0]
    ecol = edge_index[1]
    deg_part = _deg(erow, edge_attr)
    dis_col, y0, t0 = _tca(xp, deg_part, W1[0])
    p1 = _prop(y0, erow, ecol, edge_attr)
    y1, t1 = _tcb(t0, p1, dis_col, W1[1], b1.reshape(1, F), W2[0])
    p2 = _prop(y1, erow, ecol, edge_attr)
    y2, t2 = _tcb(t1, p2, dis_col, W2[1], b2.reshape(1, F), W3[0])
    p3 = _prop(y2, erow, ecol, edge_attr)
    out = _tcc(t2, p3, dis_col, W3[1], b3.reshape(1, F), batp,
               Wl1, bl1.reshape(1, 32), Wl2, bl2.reshape(1, NCLS))
    return out
